# baseline (speedup 1.0000x reference)
import functools

import jax
import jax.numpy as jnp
from jax import lax
from jax.experimental import pallas as pl
from jax.experimental.pallas import tpu as pltpu
from jax.experimental.pallas import tpu_sc as plsc


def _sc_gather_rows(step, pat, tpe, qpe):
    B = pat.shape[0]
    TD = tpe.shape[1]
    V, QD = qpe.shape
    mesh = plsc.VectorSubcoreMesh(core_axis_name="c", subcore_axis_name="s",
                                  num_cores=1)

    @functools.partial(
        pl.kernel,
        mesh=mesh,
        out_type=[
            jax.ShapeDtypeStruct((1, TD), jnp.float32),
            jax.ShapeDtypeStruct((B, QD), jnp.float32),
        ],
        scratch_types=[
            pltpu.VMEM((1,), jnp.int32),
            pltpu.VMEM((B,), jnp.int32),
            pltpu.VMEM((1, TD), jnp.float32),
            pltpu.VMEM((B, QD), jnp.float32),
            pltpu.SemaphoreType.DMA,
        ],
    )
    def gather(step_hbm, pat_hbm, tpe_hbm, qpe_hbm, trow_hbm, rows_hbm,
               step_v, idx_v, trow_v, rows_v, sem):
        wid = lax.axis_index("s")

        @pl.when(wid == 0)
        def _():
            pltpu.sync_copy(step_hbm, step_v)
            pltpu.sync_copy(pat_hbm, idx_v)
            idx_v[...] = lax.rem(idx_v[...], V)
            pltpu.async_copy(tpe_hbm.at[step_v], trow_v, sem).wait()
            pltpu.async_copy(qpe_hbm.at[idx_v], rows_v, sem).wait()
            pltpu.sync_copy(trow_v, trow_hbm)
            pltpu.sync_copy(rows_v, rows_hbm)

    return gather(step, pat, tpe, qpe)


def _body(x_ref, trow_ref, spe_ref, qrow_ref, o_ref):
    td = trow_ref.shape[2]
    sd = spe_ref.shape[2]
    t_row = trow_ref[0, 0, :]
    q_row = qrow_ref[0, 0, :]
    o_ref[..., :td] = x_ref[..., :td] + t_row[None, None, None, :]
    o_ref[..., td:td + sd] = x_ref[..., td:td + sd] + spe_ref[...][None]
    o_ref[..., td + sd:] = x_ref[..., td + sd:] + q_row[None, None, None, :]


def kernel(x, temporal_step, sequence_pattern, temporal_pe, spatial_pe, sequence_pe):
    B, H, W, D = x.shape
    TD = temporal_pe.shape[1]
    SD = spatial_pe.shape[2]
    QD = sequence_pe.shape[1]
    RB = 64
    R = H // RB

    step = jnp.asarray(temporal_step, jnp.int32).reshape(1)
    pat = jnp.asarray(sequence_pattern, jnp.int32)

    t_row, seq_rows = _sc_gather_rows(step, pat, temporal_pe, sequence_pe)
    t_row = t_row.reshape(1, 1, TD)
    seq_rows = seq_rows.reshape(B, 1, QD)

    return pl.pallas_call(
        _body,
        grid=(R, B),
        in_specs=[
            pl.BlockSpec((1, RB, W, D), lambda r, b: (b, r, 0, 0)),
            pl.BlockSpec((1, 1, TD), lambda r, b: (0, 0, 0)),
            pl.BlockSpec((RB, W, SD), lambda r, b: (r, 0, 0)),
            pl.BlockSpec((1, 1, QD), lambda r, b: (b, 0, 0)),
        ],
        out_specs=pl.BlockSpec((1, RB, W, D), lambda r, b: (b, r, 0, 0)),
        out_shape=jax.ShapeDtypeStruct(x.shape, x.dtype),
        compiler_params=pltpu.CompilerParams(
            dimension_semantics=("parallel", "parallel"),
        ),
    )(x, t_row, spatial_pe, seq_rows)


# speedup vs baseline: 1.0121x; 1.0121x over previous
import functools

import jax
import jax.numpy as jnp
from jax import lax
from jax.experimental import pallas as pl
from jax.experimental.pallas import tpu as pltpu
from jax.experimental.pallas import tpu_sc as plsc


def _sc_gather_rows(pat, table):
    B = pat.shape[0]
    V, QD = table.shape
    mesh = plsc.VectorSubcoreMesh(core_axis_name="c", subcore_axis_name="s",
                                  num_cores=1)

    @functools.partial(
        pl.kernel,
        mesh=mesh,
        out_type=jax.ShapeDtypeStruct((B, QD), jnp.float32),
        scratch_types=[
            pltpu.VMEM((B,), jnp.int32),
            pltpu.VMEM((B, QD), jnp.float32),
            pltpu.SemaphoreType.DMA,
        ],
    )
    def gather(idx_hbm, table_hbm, out_hbm, idx_v, rows_v, sem):
        wid = lax.axis_index("s")

        @pl.when(wid == 0)
        def _():
            pltpu.sync_copy(idx_hbm, idx_v)
            idx_v[...] = lax.rem(idx_v[...], V)
            pltpu.async_copy(table_hbm.at[idx_v], rows_v, sem).wait()
            pltpu.sync_copy(rows_v, out_hbm)

    return gather(pat, table)


def _body(step_ref, x_ref, tpe_ref, spe_ref, qrow_ref, o_ref):
    s = step_ref[0]
    td = tpe_ref.shape[1]
    sd = spe_ref.shape[2]
    t_row = tpe_ref[s, :]
    q_row = qrow_ref[0, 0, :]
    o_ref[..., :td] = x_ref[..., :td] + t_row[None, None, None, :]
    o_ref[..., td:td + sd] = x_ref[..., td:td + sd] + spe_ref[...][None]
    o_ref[..., td + sd:] = x_ref[..., td + sd:] + q_row[None, None, None, :]


def kernel(x, temporal_step, sequence_pattern, temporal_pe, spatial_pe, sequence_pe):
    B, H, W, D = x.shape
    SD = spatial_pe.shape[2]
    QD = sequence_pe.shape[1]
    RB = 64
    R = H // RB

    step = jnp.asarray(temporal_step, jnp.int32).reshape(1)
    pat = jnp.asarray(sequence_pattern, jnp.int32)

    seq_rows = _sc_gather_rows(pat, sequence_pe)
    seq_rows = seq_rows.reshape(B, 1, QD)

    grid_spec = pltpu.PrefetchScalarGridSpec(
        num_scalar_prefetch=1,
        grid=(R, B),
        in_specs=[
            pl.BlockSpec((1, RB, W, D), lambda r, b, *_: (b, r, 0, 0)),
            pl.BlockSpec(temporal_pe.shape, lambda r, b, *_: (0, 0)),
            pl.BlockSpec((RB, W, SD), lambda r, b, *_: (r, 0, 0)),
            pl.BlockSpec((1, 1, QD), lambda r, b, *_: (b, 0, 0)),
        ],
        out_specs=pl.BlockSpec((1, RB, W, D), lambda r, b, *_: (b, r, 0, 0)),
    )
    return pl.pallas_call(
        _body,
        grid_spec=grid_spec,
        out_shape=jax.ShapeDtypeStruct(x.shape, x.dtype),
        compiler_params=pltpu.CompilerParams(
            dimension_semantics=("parallel", "parallel"),
        ),
    )(step, x, temporal_pe, spatial_pe, seq_rows)
